# Initial kernel scaffold; baseline (speedup 1.0000x reference)
import jax
import jax.numpy as jnp
from jax.experimental import pallas as pl


def kernel(x, weight, gamma, beta, running_mean, running_var):
    raise NotImplementedError("write your pallas kernel here")


# speedup vs baseline: 6.7334x; 6.7334x over previous
import functools

import jax
import jax.numpy as jnp
from jax.experimental import pallas as pl
from jax.experimental.pallas import tpu as pltpu


def _round_up(x, n):
    return ((x + n - 1) // n) * n


def _conv_taps_kernel(x_ref, halo_ref, w_ref, shift_ref, o_ref, *, ntaps,
                      offsets, cin, tm):
    xcat = jnp.concatenate([x_ref[...], halo_ref[...]], axis=0)
    acc = jnp.zeros(o_ref.shape, jnp.float32)
    for t in range(ntaps):
        d = offsets[t]
        acc += jnp.dot(xcat[d:d + tm, :], w_ref[t * cin:(t + 1) * cin, :],
                       preferred_element_type=jnp.float32)
    o_ref[...] = jnp.maximum(acc + shift_ref[...], 0.0)


@jax.jit
def _conv_bn_relu(x, weight, gamma, beta, running_mean, running_var):
    n, cin, h, w = x.shape
    cout = weight.shape[0]
    eps = 1e-5
    hp, wp = h + 2, w + 2
    pix = hp * wp
    m = n * pix

    xt = jnp.transpose(x, (0, 2, 3, 1)).astype(jnp.bfloat16)
    xt = jnp.pad(xt, ((0, 0), (1, 1), (1, 1), (0, 0)))
    xflat = xt.reshape(m, cin)

    scale = gamma / jnp.sqrt(running_var + eps)
    shift = (beta - running_mean * scale).reshape(1, cout)
    wt = (weight * scale[:, None, None, None]).astype(jnp.bfloat16)
    wt = jnp.transpose(wt, (2, 3, 1, 0)).reshape(9 * cin, cout)

    cout_pad = _round_up(cout, 128)
    if cout_pad != cout:
        wt = jnp.pad(wt, ((0, 0), (0, cout_pad - cout)))
        shift = jnp.pad(shift, ((0, 0), (0, cout_pad - cout)))

    offsets = tuple(r * wp + c for r in range(3) for c in range(3))
    halo = max(128, _round_up(offsets[-1] + 1, 128))

    tm = min(2048, _round_up(m, 128))
    m_pad = _round_up(m, tm)
    xflat = jnp.pad(xflat, ((0, m_pad + halo - m), (0, 0)))

    grid = (m_pad // tm,)
    body = functools.partial(_conv_taps_kernel, ntaps=9, offsets=offsets,
                             cin=cin, tm=tm)
    out = pl.pallas_call(
        body,
        out_shape=jax.ShapeDtypeStruct((m_pad, cout_pad), jnp.float32),
        grid=grid,
        in_specs=[
            pl.BlockSpec((tm, cin), lambda i: (i, 0)),
            pl.BlockSpec((halo, cin),
                         lambda i, _tb=tm // halo: (i * _tb + _tb, 0)),
            pl.BlockSpec((9 * cin, cout_pad), lambda i: (0, 0)),
            pl.BlockSpec((1, cout_pad), lambda i: (0, 0)),
        ],
        out_specs=pl.BlockSpec((tm, cout_pad), lambda i: (i, 0)),
        compiler_params=pltpu.CompilerParams(
            dimension_semantics=("parallel",),
        ),
    )(xflat, xflat, wt, shift)

    out = out[:m, :cout].reshape(n, hp, wp, cout)[:, :h, :w, :]
    return jnp.transpose(out, (0, 3, 1, 2))


def kernel(x, weight, gamma, beta, running_mean, running_var):
    return _conv_bn_relu(x, weight, gamma, beta, running_mean, running_var)


# speedup vs baseline: 10.3376x; 1.5353x over previous
import functools

import jax
import jax.numpy as jnp
from jax.experimental import pallas as pl
from jax.experimental.pallas import tpu as pltpu


def _round_up(x, n):
    return ((x + n - 1) // n) * n


def _conv_t_kernel(x_ref, w_ref, m0_ref, m2_ref, s_ref, o_ref, *, wdim, q_total,
                   lt):
    xv = x_ref[0]
    sh = s_ref[...]
    for q0 in range(0, q_total, lt):
        m0 = m0_ref[:, q0:q0 + lt]
        m2 = m2_ref[:, q0:q0 + lt]
        acc = jnp.zeros((o_ref.shape[1], lt), jnp.float32)
        for t in range(9):
            r, c = divmod(t, 3)
            d = r * wdim + c + q0
            xs = xv[:, d:d + lt]
            if c == 0:
                xs = xs * m0
            elif c == 2:
                xs = xs * m2
            acc += jnp.dot(w_ref[t], xs, preferred_element_type=jnp.float32)
        o_ref[0, :, q0:q0 + lt] = jnp.maximum(acc + sh, 0.0)


@jax.jit
def _conv_bn_relu(x, weight, gamma, beta, running_mean, running_var):
    n, cin, h, w = x.shape
    cout = weight.shape[0]
    eps = 1e-5
    q = h * w
    p0 = w + 1
    lanes = _round_up(p0 + (h + 1) * w + w + 1, 128)

    xf = x.reshape(n, cin, q).astype(jnp.bfloat16)
    xf = jnp.pad(xf, ((0, 0), (0, 0), (p0, lanes - p0 - q)))

    scale = gamma / jnp.sqrt(running_var + eps)
    shift = (beta - running_mean * scale).reshape(cout, 1)
    wt = (weight * scale[:, None, None, None]).astype(jnp.bfloat16)
    wt = jnp.transpose(wt, (2, 3, 0, 1)).reshape(9, cout, cin)

    wpos = jnp.arange(q, dtype=jnp.int32) % w
    m0 = (wpos != 0).astype(jnp.bfloat16).reshape(1, q)
    m2 = (wpos != w - 1).astype(jnp.bfloat16).reshape(1, q)

    lt = q
    for cand in (448, 512, 384, 256):
        if q % cand == 0:
            lt = cand
            break

    body = functools.partial(_conv_t_kernel, wdim=w, q_total=q, lt=lt)
    out = pl.pallas_call(
        body,
        out_shape=jax.ShapeDtypeStruct((n, cout, q), jnp.float32),
        grid=(n,),
        in_specs=[
            pl.BlockSpec((1, cin, lanes), lambda i: (i, 0, 0)),
            pl.BlockSpec((9, cout, cin), lambda i: (0, 0, 0)),
            pl.BlockSpec((1, q), lambda i: (0, 0)),
            pl.BlockSpec((1, q), lambda i: (0, 0)),
            pl.BlockSpec((cout, 1), lambda i: (0, 0)),
        ],
        out_specs=pl.BlockSpec((1, cout, q), lambda i: (i, 0, 0)),
        compiler_params=pltpu.CompilerParams(
            dimension_semantics=("parallel",),
        ),
    )(xf, wt, m0, m2, shift)

    return out.reshape(n, cout, h, w)


def kernel(x, weight, gamma, beta, running_mean, running_var):
    return _conv_bn_relu(x, weight, gamma, beta, running_mean, running_var)
